# baseline (speedup 1.0000x reference)
import functools

import jax
import jax.numpy as jnp
from jax import lax
from jax.experimental import pallas as pl
from jax.experimental.pallas import tpu as pltpu
from jax.experimental.pallas import tpu_sc as plsc

S = 7
B = 2
C = 20
LEN = 5 * B + C
BS = 64
N_CELLS = BS * S * S
L = 16
NV = N_CELLS // L
NS = 16
K = -(-NV // NS)

_f32 = jnp.float32


def _cell_losses(pvm, tvm):
    off = lax.iota(jnp.int32, L) * LEN

    def pcol(c):
        return plsc.load_gather(pvm, [off + c])

    def tcol(c):
        return plsc.load_gather(tvm, [off + c])

    tc4 = tcol(4)
    tc9 = tcol(9)
    pc0 = pcol(4)
    pc1 = pcol(9)
    coo = tc4 > _f32(0.0)
    coo_f = jnp.where(coo, _f32(1.0), _f32(0.0))
    noo_f = jnp.where(tc4 == _f32(0.0), _f32(1.0), _f32(0.0))

    d0 = pc0 - tc4
    d1 = pc1 - tc9
    noo_loss = noo_f * (d0 * d0 + d1 * d1)

    tx, ty, tw, th = tcol(0), tcol(1), tcol(2), tcol(3)
    t1x = tx / _f32(S) - _f32(0.5) * tw
    t2x = tx / _f32(S) + _f32(0.5) * tw
    t1y = ty / _f32(S) - _f32(0.5) * th
    t2y = ty / _f32(S) + _f32(0.5) * th
    a2 = (t2x - t1x) * (t2y - t1y)

    def iou(px, py, pw, ph):
        p1x = px / _f32(S) - _f32(0.5) * pw
        p2x = px / _f32(S) + _f32(0.5) * pw
        p1y = py / _f32(S) - _f32(0.5) * ph
        p2y = py / _f32(S) + _f32(0.5) * ph
        wx = jnp.maximum(jnp.minimum(p2x, t2x) - jnp.maximum(p1x, t1x), _f32(0.0))
        wy = jnp.maximum(jnp.minimum(p2y, t2y) - jnp.maximum(p1y, t1y), _f32(0.0))
        inter = wx * wy
        a1 = (p2x - p1x) * (p2y - p1y)
        denom = a1 + a2 - inter
        safe = jnp.where(coo, denom, _f32(1.0))
        return inter / safe

    iou0 = iou(pcol(0), pcol(1), pcol(2), pcol(3))
    iou1 = iou(pcol(5), pcol(6), pcol(7), pcol(8))
    take1 = iou1 > iou0
    max_iou = jnp.maximum(iou0, iou1)
    resp_c = jnp.where(take1, pc1, pc0)
    dc = resp_c - max_iou
    contain_loss = coo_f * (dc * dc)

    cls = jnp.zeros((L,), _f32)
    for c in range(C):
        d = pcol(10 + c) - tcol(10 + c)
        cls = cls + d * d
    class_loss = coo_f * cls

    return contain_loss + _f32(0.5) * noo_loss + class_loss


def _sc_body(pred_hbm, tgt_hbm, out_hbm, pvm, tvm, accvm, redvm, shared):
    sid = lax.axis_index("s")
    accvm[...] = jnp.zeros((L,), _f32)

    for k in range(K):
        v = sid + NS * k

        @pl.when(v < NV)
        def _():
            base = v * (L * LEN)
            pltpu.sync_copy(pred_hbm.at[pl.ds(base, L * LEN)], pvm)
            pltpu.sync_copy(tgt_hbm.at[pl.ds(base, L * LEN)], tvm)
            accvm[...] = accvm[...] + _cell_losses(pvm, tvm)

    pltpu.sync_copy(accvm, shared.at[sid])
    plsc.subcore_barrier()

    @pl.when(sid == 0)
    def _():
        pltpu.sync_copy(shared, redvm)
        t = jnp.zeros((L,), _f32)
        for i in range(NS):
            t = t + redvm[i, :]
        total = jnp.sum(t) * _f32(1.0 / BS)
        accvm[...] = jnp.full((L,), total, _f32)
        pltpu.sync_copy(accvm, out_hbm)


_mesh = plsc.VectorSubcoreMesh(
    core_axis_name="c", subcore_axis_name="s", num_cores=1)

_sc_yolo = functools.partial(
    pl.kernel,
    out_type=jax.ShapeDtypeStruct((L,), _f32),
    mesh=_mesh,
    compiler_params=pltpu.CompilerParams(
        needs_layout_passes=False, use_tc_tiling_on_sc=False),
    scratch_types=[
        pltpu.VMEM((L * LEN,), _f32),
        pltpu.VMEM((L * LEN,), _f32),
        pltpu.VMEM((L,), _f32),
        pltpu.VMEM((NS, L), _f32),
        pltpu.VMEM_SHARED((NS, L), _f32),
    ],
)(_sc_body)


def kernel(prediction, target):
    out = _sc_yolo(prediction.reshape(-1), target.reshape(-1))
    return out[0]


# speedup vs baseline: 1.5153x; 1.5153x over previous
import functools

import jax
import jax.numpy as jnp
from jax import lax
from jax.experimental import pallas as pl
from jax.experimental.pallas import tpu as pltpu
from jax.experimental.pallas import tpu_sc as plsc

S = 7
B = 2
C = 20
LEN = 5 * B + C
BS = 64
N_CELLS = BS * S * S
L = 16
NS = 16
CPT = N_CELLS // NS
FPT = CPT * LEN
FULL = CPT // L
TAIL = CPT - FULL * L

_f32 = jnp.float32


def _accum_losses(pvm, tvm, off, wt, accvm):
    def pcol(c):
        return plsc.load_gather(pvm, [off + c])

    def tcol(c):
        return plsc.load_gather(tvm, [off + c])

    tc4 = tcol(4)
    tc9 = tcol(9)
    pc0 = pcol(4)
    pc1 = pcol(9)

    noo_f = jnp.where(tc4 == _f32(0.0), _f32(1.0), _f32(0.0))
    d0 = pc0 - tc4
    d1 = pc1 - tc9
    noo = _f32(0.5) * noo_f * (d0 * d0 + d1 * d1)
    if wt is not None:
        noo = jnp.where(wt, noo, _f32(0.0))
    accvm[...] = accvm[...] + noo

    @pl.when(jnp.max(tc4) > _f32(0.0))
    def _():
        coo = tc4 > _f32(0.0)
        coo_f = jnp.where(coo, _f32(1.0), _f32(0.0))

        tx, ty, tw, th = tcol(0), tcol(1), tcol(2), tcol(3)
        t1x = tx / _f32(S) - _f32(0.5) * tw
        t2x = tx / _f32(S) + _f32(0.5) * tw
        t1y = ty / _f32(S) - _f32(0.5) * th
        t2y = ty / _f32(S) + _f32(0.5) * th
        a2 = (t2x - t1x) * (t2y - t1y)

        def iou(px, py, pw, ph):
            p1x = px / _f32(S) - _f32(0.5) * pw
            p2x = px / _f32(S) + _f32(0.5) * pw
            p1y = py / _f32(S) - _f32(0.5) * ph
            p2y = py / _f32(S) + _f32(0.5) * ph
            wx = jnp.maximum(
                jnp.minimum(p2x, t2x) - jnp.maximum(p1x, t1x), _f32(0.0))
            wy = jnp.maximum(
                jnp.minimum(p2y, t2y) - jnp.maximum(p1y, t1y), _f32(0.0))
            inter = wx * wy
            a1 = (p2x - p1x) * (p2y - p1y)
            denom = a1 + a2 - inter
            safe = jnp.where(coo, denom, _f32(1.0))
            return inter / safe

        iou0 = iou(pcol(0), pcol(1), pcol(2), pcol(3))
        iou1 = iou(pcol(5), pcol(6), pcol(7), pcol(8))
        max_iou = jnp.maximum(iou0, iou1)
        resp_c = jnp.where(iou1 > iou0, pc1, pc0)
        dc = resp_c - max_iou
        contain = dc * dc

        cls = jnp.zeros((L,), _f32)
        for c in range(C):
            d = pcol(10 + c) - tcol(10 + c)
            cls = cls + d * d

        obj = coo_f * (contain + cls)
        if wt is not None:
            obj = jnp.where(wt, obj, _f32(0.0))
        accvm[...] = accvm[...] + obj


def _sc_body(pred_hbm, tgt_hbm, out_hbm, pvm, tvm, accvm, redvm, shared,
             sem_p, sem_t):
    sid = lax.axis_index("s")
    base = sid * FPT
    cp = pltpu.async_copy(pred_hbm.at[pl.ds(base, FPT)], pvm, sem_p)
    ct = pltpu.async_copy(tgt_hbm.at[pl.ds(base, FPT)], tvm, sem_t)
    cp.wait()
    ct.wait()

    accvm[...] = jnp.zeros((L,), _f32)
    lane = lax.iota(jnp.int32, L)
    for k in range(FULL + 1):
        if k < FULL:
            off = (lane + k * L) * LEN
            wt = None
        else:
            off = (jnp.minimum(lane, TAIL - 1) + k * L) * LEN
            wt = lane < TAIL
        _accum_losses(pvm, tvm, off, wt, accvm)

    pltpu.sync_copy(accvm, shared.at[sid])
    plsc.subcore_barrier()

    @pl.when(sid == 0)
    def _():
        pltpu.sync_copy(shared, redvm)
        t = jnp.zeros((L,), _f32)
        for i in range(NS):
            t = t + redvm[i, :]
        total = jnp.sum(t) * _f32(1.0 / BS)
        accvm[...] = jnp.full((L,), total, _f32)
        pltpu.sync_copy(accvm, out_hbm)


_mesh = plsc.VectorSubcoreMesh(
    core_axis_name="c", subcore_axis_name="s", num_cores=1)

_sc_yolo = functools.partial(
    pl.kernel,
    out_type=jax.ShapeDtypeStruct((L,), _f32),
    mesh=_mesh,
    compiler_params=pltpu.CompilerParams(
        needs_layout_passes=False, use_tc_tiling_on_sc=False),
    scratch_types=[
        pltpu.VMEM((FPT,), _f32),
        pltpu.VMEM((FPT,), _f32),
        pltpu.VMEM((L,), _f32),
        pltpu.VMEM((NS, L), _f32),
        pltpu.VMEM_SHARED((NS, L), _f32),
        pltpu.SemaphoreType.DMA,
        pltpu.SemaphoreType.DMA,
    ],
)(_sc_body)


def kernel(prediction, target):
    out = _sc_yolo(prediction.reshape(-1), target.reshape(-1))
    return out[0]


# speedup vs baseline: 1.5811x; 1.0434x over previous
import functools

import jax
import jax.numpy as jnp
from jax import lax
from jax.experimental import pallas as pl
from jax.experimental.pallas import tpu as pltpu
from jax.experimental.pallas import tpu_sc as plsc

S = 7
B = 2
C = 20
LEN = 5 * B + C
BS = 64
N_CELLS = BS * S * S
L = 16
NS = 16
CPT = N_CELLS // NS
FPT = CPT * LEN
FULL = CPT // L
TAIL = CPT - FULL * L

_f32 = jnp.float32


def _accum_losses(pvm, tvm, off, wt, accvm):
    def pcol(c):
        return plsc.load_gather(pvm, [off + c])

    def tcol(c):
        return plsc.load_gather(tvm, [off + c])

    tc4 = tcol(4)
    tc9 = tcol(9)
    pc0 = pcol(4)
    pc1 = pcol(9)

    noo_f = jnp.where(tc4 == _f32(0.0), _f32(1.0), _f32(0.0))
    d0 = pc0 - tc4
    d1 = pc1 - tc9
    noo = _f32(0.5) * noo_f * (d0 * d0 + d1 * d1)
    if wt is not None:
        noo = jnp.where(wt, noo, _f32(0.0))
    accvm[...] = accvm[...] + noo

    @pl.when(jnp.max(tc4) > _f32(0.0))
    def _():
        coo = tc4 > _f32(0.0)
        coo_f = jnp.where(coo, _f32(1.0), _f32(0.0))

        tx, ty, tw, th = tcol(0), tcol(1), tcol(2), tcol(3)
        t1x = tx / _f32(S) - _f32(0.5) * tw
        t2x = tx / _f32(S) + _f32(0.5) * tw
        t1y = ty / _f32(S) - _f32(0.5) * th
        t2y = ty / _f32(S) + _f32(0.5) * th
        a2 = (t2x - t1x) * (t2y - t1y)

        def iou(px, py, pw, ph):
            p1x = px / _f32(S) - _f32(0.5) * pw
            p2x = px / _f32(S) + _f32(0.5) * pw
            p1y = py / _f32(S) - _f32(0.5) * ph
            p2y = py / _f32(S) + _f32(0.5) * ph
            wx = jnp.maximum(
                jnp.minimum(p2x, t2x) - jnp.maximum(p1x, t1x), _f32(0.0))
            wy = jnp.maximum(
                jnp.minimum(p2y, t2y) - jnp.maximum(p1y, t1y), _f32(0.0))
            inter = wx * wy
            a1 = (p2x - p1x) * (p2y - p1y)
            denom = a1 + a2 - inter
            safe = jnp.where(coo, denom, _f32(1.0))
            return inter / safe

        iou0 = iou(pcol(0), pcol(1), pcol(2), pcol(3))
        iou1 = iou(pcol(5), pcol(6), pcol(7), pcol(8))
        max_iou = jnp.maximum(iou0, iou1)
        resp_c = jnp.where(iou1 > iou0, pc1, pc0)
        dc = resp_c - max_iou
        contain = dc * dc

        cls = jnp.zeros((L,), _f32)
        for c in range(C):
            d = pcol(10 + c) - tcol(10 + c)
            cls = cls + d * d

        obj = coo_f * (contain + cls)
        if wt is not None:
            obj = jnp.where(wt, obj, _f32(0.0))
        accvm[...] = accvm[...] + obj


def _sc_body(pred_hbm, tgt_hbm, out_hbm, pvm, tvm, accvm, redvm, shared,
             sem_p, sem_t):
    sid = lax.axis_index("s")
    base = sid * FPT
    cp = pltpu.async_copy(pred_hbm.at[pl.ds(base, FPT)], pvm, sem_p)
    ct = pltpu.async_copy(tgt_hbm.at[pl.ds(base, FPT)], tvm, sem_t)
    cp.wait()
    ct.wait()

    accvm[...] = jnp.zeros((L,), _f32)
    lane = lax.iota(jnp.int32, L)

    def vec_body(k, carry):
        off = (lane + k * L) * LEN
        _accum_losses(pvm, tvm, off, None, accvm)
        return carry

    lax.fori_loop(0, FULL, vec_body, 0)
    off = (jnp.minimum(lane, TAIL - 1) + FULL * L) * LEN
    _accum_losses(pvm, tvm, off, lane < TAIL, accvm)

    pltpu.sync_copy(accvm, shared.at[sid])
    plsc.subcore_barrier()

    @pl.when(sid == 0)
    def _():
        pltpu.sync_copy(shared, redvm)
        t = jnp.zeros((L,), _f32)
        for i in range(NS):
            t = t + redvm[i, :]
        total = jnp.sum(t) * _f32(1.0 / BS)
        accvm[...] = jnp.full((L,), total, _f32)
        pltpu.sync_copy(accvm, out_hbm)


_mesh = plsc.VectorSubcoreMesh(
    core_axis_name="c", subcore_axis_name="s", num_cores=1)

_sc_yolo = functools.partial(
    pl.kernel,
    out_type=jax.ShapeDtypeStruct((L,), _f32),
    mesh=_mesh,
    compiler_params=pltpu.CompilerParams(
        needs_layout_passes=False, use_tc_tiling_on_sc=False),
    scratch_types=[
        pltpu.VMEM((FPT,), _f32),
        pltpu.VMEM((FPT,), _f32),
        pltpu.VMEM((L,), _f32),
        pltpu.VMEM((NS, L), _f32),
        pltpu.VMEM_SHARED((NS, L), _f32),
        pltpu.SemaphoreType.DMA,
        pltpu.SemaphoreType.DMA,
    ],
)(_sc_body)


def kernel(prediction, target):
    out = _sc_yolo(prediction.reshape(-1), target.reshape(-1))
    return out[0]
